# Initial kernel scaffold; baseline (speedup 1.0000x reference)
import jax
import jax.numpy as jnp
from jax.experimental import pallas as pl


def kernel(x, weight, emb_bias):
    raise NotImplementedError("write your pallas kernel here")


# speedup vs baseline: 1.4736x; 1.4736x over previous
import functools

import jax
import jax.numpy as jnp
from jax import lax
from jax.experimental import pallas as pl
from jax.experimental.pallas import tpu as pltpu
from jax.experimental.pallas import tpu_sc as plsc

B = 16384
H = 50
D = 64
NC, NS = 2, 16
NW = NC * NS
EPW = B // NW
CE = 2
PB = 56 - 4
CPW = CE * PB
NCH = EPW // CE
NB = 4
RU = 10


def _body(x_ref, w_ref, b_ref, o_ref, idx_v, bias_v, out_v,
          rows0, rows1, rows2, rows3, sem0, sem1, sem2, sem3):
    rows = (rows0, rows1, rows2, rows3)
    sems = (sem0, sem1, sem2, sem3)
    wid = lax.axis_index("s") * NC + lax.axis_index("c")

    pltpu.sync_copy(x_ref.at[wid], idx_v)
    pltpu.sync_copy(b_ref, bias_v)
    bias_vec = [bias_v[pl.ds(k * 16, 16)] for k in range(4)]
    inv_h = jnp.float32(1.0 / H)

    def start_gather(c, b):
        pltpu.async_copy(w_ref.at[idx_v.at[c]], rows[b], sems[b])

    for b in range(NB):
        start_gather(b, b)

    @pl.loop(0, NCH, step=NB)
    def _chunks(j):
        for b in range(NB):
            c = j + b
            pltpu.make_async_copy(w_ref.at[idx_v.at[c]], rows[b], sems[b]).wait()
            for e in range(CE):
                base = e * PB

                def rbody(it, acc, _b=b, _base=base):
                    r0 = _base + it * RU
                    a = list(acc)
                    for u in range(RU):
                        for k in range(4):
                            a[k] = a[k] + rows[_b][r0 + u, pl.ds(k * 16, 16)]
                    return tuple(a)

                z = jnp.zeros((16,), jnp.float32)
                acc = lax.fori_loop(0, H // RU, rbody, (z, z, z, z))
                orow = c * CE + e
                for k in range(4):
                    out_v[orow, pl.ds(k * 16, 16)] = (
                        acc[k] * inv_h + bias_vec[k])

            @pl.when(c + NB < NCH)
            def _():
                start_gather(c + NB, b)

    pltpu.sync_copy(out_v, o_ref.at[pl.ds(wid * EPW, EPW)])


@functools.partial(jax.jit, static_argnames=())
def _emb_sum(x3, weight, emb_bias):
    mesh = plsc.VectorSubcoreMesh(core_axis_name="c", subcore_axis_name="s")
    f = pl.kernel(
        _body,
        out_type=jax.ShapeDtypeStruct((B, D), jnp.float32),
        mesh=mesh,
        scratch_types=[
            pltpu.VMEM((NCH, CPW), jnp.int32),
            pltpu.VMEM((D,), jnp.float32),
            pltpu.VMEM((EPW, D), jnp.float32),
        ] + [pltpu.VMEM((CPW, D), jnp.float32) for _ in range(NB)]
          + [pltpu.SemaphoreType.DMA for _ in range(NB)],
        compiler_params=pltpu.CompilerParams(use_tc_tiling_on_sc=False),
    )
    return f(x3, weight, emb_bias)


def kernel(x, weight, emb_bias):
    x4 = x.astype(jnp.int32).reshape(NW, NCH, CE, H)
    xp = jnp.pad(x4, ((0, 0), (0, 0), (0, 0), (0, PB - H)))
    x3 = xp.reshape(NW, NCH, CPW)
    return _emb_sum(x3, weight, emb_bias)


# speedup vs baseline: 2.7721x; 1.8812x over previous
import functools

import jax
import jax.numpy as jnp
from jax import lax
from jax.experimental import pallas as pl
from jax.experimental.pallas import tpu as pltpu
from jax.experimental.pallas import tpu_sc as plsc

B = 16384
H = 50
D = 64
NC, NS = 2, 16
NW = NC * NS
EPW = B // NW
IPW = EPW * H
CE = 4
CPW = CE * H
NCH = EPW // CE
SPLIT = (40, 40, 40, 40, 40)
NB = 4
RU = 10


def _body(x_ref, w_ref, b_ref, o_ref, idx_v, bias_v, out_v,
          rows0, rows1, rows2, rows3, sem0, sem1, sem2, sem3):
    rows = (rows0, rows1, rows2, rows3)
    sems = (sem0, sem1, sem2, sem3)
    wid = lax.axis_index("s") * NC + lax.axis_index("c")

    pltpu.sync_copy(x_ref.at[wid], idx_v)
    pltpu.sync_copy(b_ref, bias_v)
    bias_vec = [bias_v[pl.ds(k * 16, 16)] for k in range(4)]
    inv_h = jnp.float32(1.0 / H)

    def start_gathers(c, b):
        off = 0
        for n in SPLIT:
            pltpu.async_copy(
                w_ref.at[idx_v.at[pl.ds(c * CPW + off, n)]],
                rows[b].at[pl.ds(off, n)], sems[b])
            off += n

    def wait_gathers(c, b):
        off = 0
        for n in SPLIT:
            pltpu.make_async_copy(
                w_ref.at[idx_v.at[pl.ds(c * CPW + off, n)]],
                rows[b].at[pl.ds(off, n)], sems[b]).wait()
            off += n

    for b in range(NB):
        start_gathers(b, b)

    @pl.loop(0, NCH, step=NB)
    def _chunks(j):
        for b in range(NB):
            c = j + b
            wait_gathers(c, b)
            for e in range(CE):
                base = e * H

                def rbody(it, acc, _b=b, _base=base):
                    r0 = _base + it * RU
                    a = list(acc)
                    for u in range(RU):
                        for k in range(4):
                            a[k] = a[k] + rows[_b][r0 + u, pl.ds(k * 16, 16)]
                    return tuple(a)

                z = jnp.zeros((16,), jnp.float32)
                acc = lax.fori_loop(0, H // RU, rbody, (z, z, z, z))
                orow = c * CE + e
                for k in range(4):
                    out_v[orow, pl.ds(k * 16, 16)] = (
                        acc[k] * inv_h + bias_vec[k])

            @pl.when(c + NB < NCH)
            def _():
                start_gathers(c + NB, b)

    pltpu.sync_copy(out_v, o_ref.at[pl.ds(wid * EPW, EPW)])


@jax.jit
def _emb_sum(x3, weight, emb_bias):
    mesh = plsc.VectorSubcoreMesh(core_axis_name="c", subcore_axis_name="s")
    f = pl.kernel(
        _body,
        out_type=jax.ShapeDtypeStruct((B, D), jnp.float32),
        mesh=mesh,
        scratch_types=[
            pltpu.VMEM((IPW,), jnp.int32),
            pltpu.VMEM((D,), jnp.float32),
            pltpu.VMEM((EPW, D), jnp.float32),
        ] + [pltpu.VMEM((CPW, D), jnp.float32) for _ in range(NB)]
          + [pltpu.SemaphoreType.DMA for _ in range(NB)],
        compiler_params=pltpu.CompilerParams(use_tc_tiling_on_sc=False),
    )
    return f(x3, weight, emb_bias)


def kernel(x, weight, emb_bias):
    x3 = x.astype(jnp.int32).reshape(NW, IPW)
    return _emb_sum(x3, weight, emb_bias)
